# baseline (speedup 1.0000x reference)
import jax
import jax.numpy as jnp
from jax import lax
from jax.experimental import pallas as pl
from jax.experimental.pallas import tpu as pltpu
from jax.experimental.pallas import tpu_sc as plsc

B, C, H, W = 8, 150, 224, 224
NCH = 12
NW = 32
WPB = NW // B
H_SC = 32
H_TC = H - H_SC
NROWS = H_SC // WPB
LANES = 16
DEPTH = 4


def _tree_max(vals):
    while len(vals) > 1:
        nxt = [jnp.maximum(vals[i], vals[i + 1])
               for i in range(0, len(vals) - 1, 2)]
        if len(vals) % 2:
            nxt.append(vals[-1])
        vals = nxt
    return vals[0]


def _sc_body(a_hbm, idx_hbm, out_hbm, idx_v, buf_v, out_v, sem_in, sem_out):
    cid = lax.axis_index("c")
    sid = lax.axis_index("s")
    wid = sid * 2 + cid
    b = wid // WPB
    pr0 = H_TC + (wid % WPB) * NROWS

    pltpu.sync_copy(idx_hbm, idx_v.at[pl.ds(0, NCH)])
    pvec = idx_v[...]
    base = b * C
    for j in range(NCH):
        pltpu.async_copy(
            a_hbm.at[pvec[j] + base, pl.ds(pr0, NROWS), :],
            buf_v.at[j], sem_in)
    pltpu.make_async_copy(
        a_hbm.at[pl.ds(0, NCH), pl.ds(0, NROWS), :], buf_v, sem_in).wait()

    def rbody(r, _):
        for c in range(W // LANES):
            sl = pl.ds(c * LANES, LANES)
            acc = _tree_max([buf_v[j, r, sl] for j in range(NCH)])
            out_v[r, sl] = acc * 5.0
        return 0

    lax.fori_loop(0, NROWS, rbody, 0)
    r_out = b * H_SC + (wid % WPB) * NROWS
    pltpu.async_copy(
        out_v, out_hbm.at[pl.ds(r_out, NROWS), :], sem_out).wait()


def _tc_body(idx_ref, a_any, o_any, bufs, acc, sem, osem):
    def mk(j):
        return pltpu.make_async_copy(
            a_any.at[:, idx_ref[j], pl.ds(0, H_TC), :],
            bufs.at[j % DEPTH], sem)

    for j in range(DEPTH):
        mk(j).start()
    for j in range(NCH):
        mk(j).wait()
        if j == 0:
            acc[...] = bufs[0]
        elif j == NCH - 1:
            acc[...] = jnp.maximum(acc[...], bufs[j % DEPTH]) * 5.0
        else:
            acc[...] = jnp.maximum(acc[...], bufs[j % DEPTH])
        if j + DEPTH < NCH:
            mk(j + DEPTH).start()
    out_cp = pltpu.make_async_copy(
        acc, o_any.at[:, pl.ds(0, H_TC), :], osem)
    out_cp.start()
    out_cp.wait()


@jax.jit
def kernel(ade_objects, ade_children_mapped):
    idx = ade_children_mapped.astype(jnp.int32)
    a3 = ade_objects.reshape(B * C, H, W)

    out_tc = pl.pallas_call(
        _tc_body,
        grid_spec=pltpu.PrefetchScalarGridSpec(
            num_scalar_prefetch=1,
            grid=(),
            in_specs=[pl.BlockSpec(memory_space=pl.ANY)],
            out_specs=pl.BlockSpec(memory_space=pl.ANY),
            scratch_shapes=[
                pltpu.VMEM((DEPTH, B, H_TC, W), jnp.float32),
                pltpu.VMEM((B, H_TC, W), jnp.float32),
                pltpu.SemaphoreType.DMA,
                pltpu.SemaphoreType.DMA,
            ],
        ),
        out_shape=jax.ShapeDtypeStruct((B, H, W), jnp.float32),
    )(idx, ade_objects)

    sc_run = pl.kernel(
        _sc_body,
        jax.ShapeDtypeStruct((B * H_SC, W), jnp.float32),
        mesh=plsc.VectorSubcoreMesh(core_axis_name="c", subcore_axis_name="s"),
        scratch_types=[
            pltpu.VMEM((LANES,), jnp.int32),
            pltpu.VMEM((NCH, NROWS, W), jnp.float32),
            pltpu.VMEM((NROWS, W), jnp.float32),
            pltpu.SemaphoreType.DMA,
            pltpu.SemaphoreType.DMA,
        ],
    )
    out_sc = sc_run(a3, idx)

    return lax.dynamic_update_slice(
        out_tc, out_sc.reshape(B, H_SC, W), (0, H_TC, 0))


# speedup vs baseline: 1.0056x; 1.0056x over previous
import jax
import jax.numpy as jnp
from jax import lax
from jax.experimental import pallas as pl
from jax.experimental.pallas import tpu as pltpu
from jax.experimental.pallas import tpu_sc as plsc

B, C, H, W = 8, 150, 224, 224
NCH = 12
NW = 32
WPB = NW // B
H_SC = 16
H_TC = H - H_SC
NROWS = 8
SPB = H_SC // NROWS
SC_WORKERS = B * SPB
LANES = 16
DEPTH = 4


def _tree_max(vals):
    while len(vals) > 1:
        nxt = [jnp.maximum(vals[i], vals[i + 1])
               for i in range(0, len(vals) - 1, 2)]
        if len(vals) % 2:
            nxt.append(vals[-1])
        vals = nxt
    return vals[0]


def _sc_body(a_hbm, idx_hbm, out_hbm, idx_v, buf_v, out_v, sem_in, sem_out):
    cid = lax.axis_index("c")
    sid = lax.axis_index("s")
    wid = sid * 2 + cid

    @pl.when(wid < SC_WORKERS)
    def _():
        b = wid // SPB
        pr0 = H_TC + (wid % SPB) * NROWS

        pltpu.sync_copy(idx_hbm, idx_v.at[pl.ds(0, NCH)])
        pvec = idx_v[...]
        base = b * C
        for j in range(NCH):
            pltpu.async_copy(
                a_hbm.at[pvec[j] + base, pl.ds(pr0, NROWS), :],
                buf_v.at[j], sem_in)
        pltpu.make_async_copy(
            a_hbm.at[pl.ds(0, NCH), pl.ds(0, NROWS), :], buf_v, sem_in).wait()

        def rbody(r, _):
            for c in range(W // LANES):
                sl = pl.ds(c * LANES, LANES)
                acc = _tree_max([buf_v[j, r, sl] for j in range(NCH)])
                out_v[r, sl] = acc * 5.0
            return 0

        lax.fori_loop(0, NROWS, rbody, 0)
        r_out = b * H_SC + (wid % SPB) * NROWS
        pltpu.async_copy(
            out_v, out_hbm.at[pl.ds(r_out, NROWS), :], sem_out).wait()


def _tc_body(idx_ref, a_any, o_any, bufs, acc, sem, osem):
    def mk(j):
        return pltpu.make_async_copy(
            a_any.at[:, idx_ref[j], pl.ds(0, H_TC), :],
            bufs.at[j % DEPTH], sem)

    for j in range(DEPTH):
        mk(j).start()
    for j in range(NCH):
        mk(j).wait()
        if j == 0:
            acc[...] = bufs[0]
        elif j == NCH - 1:
            acc[...] = jnp.maximum(acc[...], bufs[j % DEPTH]) * 5.0
        else:
            acc[...] = jnp.maximum(acc[...], bufs[j % DEPTH])
        if j + DEPTH < NCH:
            mk(j + DEPTH).start()
    out_cp = pltpu.make_async_copy(
        acc, o_any.at[:, pl.ds(0, H_TC), :], osem)
    out_cp.start()
    out_cp.wait()


@jax.jit
def kernel(ade_objects, ade_children_mapped):
    idx = ade_children_mapped.astype(jnp.int32)
    a3 = ade_objects.reshape(B * C, H, W)

    sc_run = pl.kernel(
        _sc_body,
        jax.ShapeDtypeStruct((B * H_SC, W), jnp.float32),
        mesh=plsc.VectorSubcoreMesh(core_axis_name="c", subcore_axis_name="s"),
        scratch_types=[
            pltpu.VMEM((LANES,), jnp.int32),
            pltpu.VMEM((NCH, NROWS, W), jnp.float32),
            pltpu.VMEM((NROWS, W), jnp.float32),
            pltpu.SemaphoreType.DMA,
            pltpu.SemaphoreType.DMA,
        ],
    )
    out_sc = sc_run(a3, idx)

    out_tc = pl.pallas_call(
        _tc_body,
        grid_spec=pltpu.PrefetchScalarGridSpec(
            num_scalar_prefetch=1,
            grid=(),
            in_specs=[pl.BlockSpec(memory_space=pl.ANY)],
            out_specs=pl.BlockSpec(memory_space=pl.ANY),
            scratch_shapes=[
                pltpu.VMEM((DEPTH, B, H_TC, W), jnp.float32),
                pltpu.VMEM((B, H_TC, W), jnp.float32),
                pltpu.SemaphoreType.DMA,
                pltpu.SemaphoreType.DMA,
            ],
        ),
        out_shape=jax.ShapeDtypeStruct((B, H, W), jnp.float32),
    )(idx, ade_objects)

    return lax.dynamic_update_slice(
        out_tc, out_sc.reshape(B, H_SC, W), (0, H_TC, 0))
